# Initial kernel scaffold; baseline (speedup 1.0000x reference)
import jax
import jax.numpy as jnp
from jax import lax
from jax.experimental import pallas as pl
from jax.experimental.pallas import tpu as pltpu
from jax.experimental.pallas import tpu_sc as plsc

N = 10000
E = 320000
D = 128

NC = 2
NS = 16
HALF = D // NC
CW = 100
CHUNKS = E // CW
CPT = CHUNKS // NS
RPT = N // NS
RB = 125
NRB = RPT // RB
DW = 8


def _body(h_hbm, src_hbm, dst_hbm, ones_hbm, z64_hbm, z8_hbm,
          out_hbm,
          h_half, acc, deg,
          src_buf, dst_buf, msgs, ones_t,
          acc_t, h_t, out_t, deg_t):
  c = lax.axis_index("c")
  s = lax.axis_index("s")

  r0 = s * RPT
  pltpu.sync_copy(h_hbm.at[pl.ds(r0, RPT), pl.ds(c * HALF, HALF)],
                  h_half.at[pl.ds(r0, RPT)])
  pltpu.sync_copy(z64_hbm, acc.at[pl.ds(r0, RPT)])
  pltpu.sync_copy(z8_hbm, deg.at[pl.ds(r0, RPT)])
  pltpu.sync_copy(src_hbm.at[pl.ds(s * CPT, CPT)], src_buf)
  pltpu.sync_copy(dst_hbm.at[pl.ds(s * CPT, CPT)], dst_buf)
  pltpu.sync_copy(ones_hbm, ones_t)
  plsc.subcore_barrier()

  def step(j, carry):
    pltpu.sync_copy(h_half.at[src_buf.at[j]], msgs)
    pltpu.sync_copy(msgs, acc.at[dst_buf.at[j]], add=True)
    pltpu.sync_copy(ones_t, deg.at[dst_buf.at[j]], add=True)
    return carry

  lax.fori_loop(0, CPT, step, 0)
  plsc.subcore_barrier()

  def wblock(b, carry):
    row = s * RPT + b * RB
    pltpu.sync_copy(acc.at[pl.ds(row, RB)], acc_t)
    pltpu.sync_copy(h_half.at[pl.ds(row, RB)], h_t)
    pltpu.sync_copy(deg.at[pl.ds(row, RB)], deg_t)

    def wrow(i, carry2):
      flag = deg_t[i, 0] > 0.0
      m = jnp.broadcast_to(flag, (16,))
      for k in range(HALF // 16):
        av = acc_t[i, pl.ds(k * 16, 16)]
        hv = h_t[i, pl.ds(k * 16, 16)]
        out_t[i, pl.ds(k * 16, 16)] = jnp.where(m, av, hv)
      return carry2

    lax.fori_loop(0, RB, wrow, 0)
    pltpu.sync_copy(out_t,
                    out_hbm.at[pl.ds(row, RB), pl.ds(c * HALF, HALF)])
    return carry

  lax.fori_loop(0, NRB, wblock, 0)


@jax.jit
def kernel(h, h_in, edge_index):
  del h_in
  src = edge_index[0].reshape(CHUNKS, CW)
  dst = edge_index[1].reshape(CHUNKS, CW)
  ones = jnp.ones((CW, DW), jnp.float32)
  z64 = jnp.zeros((RPT, HALF), jnp.float32)
  z8 = jnp.zeros((RPT, DW), jnp.float32)

  mesh = plsc.VectorSubcoreMesh(core_axis_name="c", subcore_axis_name="s")
  run = pl.kernel(
      _body,
      out_type=jax.ShapeDtypeStruct((N, D), jnp.float32),
      mesh=mesh,
      scratch_types=[
          pltpu.VMEM_SHARED((N, HALF), jnp.float32),
          pltpu.VMEM_SHARED((N, HALF), jnp.float32),
          pltpu.VMEM_SHARED((N, DW), jnp.float32),
          pltpu.VMEM((CPT, CW), jnp.int32),
          pltpu.VMEM((CPT, CW), jnp.int32),
          pltpu.VMEM((CW, HALF), jnp.float32),
          pltpu.VMEM((CW, DW), jnp.float32),
          pltpu.VMEM((RB, HALF), jnp.float32),
          pltpu.VMEM((RB, HALF), jnp.float32),
          pltpu.VMEM((RB, HALF), jnp.float32),
          pltpu.VMEM((RB, DW), jnp.float32),
      ],
  )
  return run(h, src, dst, ones, z64, z8)


# speedup vs baseline: 6.4575x; 6.4575x over previous
import jax
import jax.numpy as jnp
from jax import lax
from jax.experimental import pallas as pl
from jax.experimental.pallas import tpu as pltpu
from jax.experimental.pallas import tpu_sc as plsc

N = 10000
E = 320000
D = 128

NC = 2
NS = 16
HALF = D // NC
CW = 100
CHUNKS = E // CW
CPT = CHUNKS // NS
BLK = 25
NBLK = CPT // BLK
RPT = N // NS
RB = 125
NRB = RPT // RB
DW = 16


def _body(hcat_hbm, src_hbm, dst_hbm, ones_hbm, z64_hbm, zdeg_hbm,
          out_hbm,
          acc, deg,
          src_buf, dst_buf, msgs, ones_t,
          acc_t, h_t, out_t, deg_t):
  c = lax.axis_index("c")
  s = lax.axis_index("s")
  hview = hcat_hbm.at[pl.ds(c * N, N)]

  r0 = s * RPT
  pltpu.sync_copy(z64_hbm, acc.at[pl.ds(r0, RPT)])
  pltpu.sync_copy(zdeg_hbm, deg.at[pl.ds(r0, RPT)])
  pltpu.sync_copy(ones_hbm, ones_t)
  plsc.subcore_barrier()

  def block(b, carry):
    base = s * CPT + b * BLK
    pltpu.sync_copy(src_hbm.at[pl.ds(base, BLK)], src_buf)
    pltpu.sync_copy(dst_hbm.at[pl.ds(base, BLK)], dst_buf)

    def step(j, carry2):
      pltpu.sync_copy(hview.at[src_buf.at[j]], msgs)
      pltpu.sync_copy(msgs, acc.at[dst_buf.at[j]], add=True)
      pltpu.sync_copy(ones_t, deg.at[dst_buf.at[j]], add=True)
      return carry2

    lax.fori_loop(0, BLK, step, 0)
    return carry

  lax.fori_loop(0, NBLK, block, 0)
  plsc.subcore_barrier()

  def wblock(b, carry):
    row = s * RPT + b * RB
    pltpu.sync_copy(acc.at[pl.ds(row, RB)], acc_t)
    pltpu.sync_copy(hview.at[pl.ds(row, RB)], h_t)
    pltpu.sync_copy(deg.at[pl.ds(row, RB)], deg_t)

    def wrow(i, carry2):
      m = deg_t[i, pl.ds(0, DW)] > 0.0
      for k in range(HALF // 16):
        av = acc_t[i, pl.ds(k * 16, 16)]
        hv = h_t[i, pl.ds(k * 16, 16)]
        out_t[i, pl.ds(k * 16, 16)] = jnp.where(m, av, hv)
      return carry2

    lax.fori_loop(0, RB, wrow, 0)
    pltpu.sync_copy(out_t,
                    out_hbm.at[pl.ds(row, RB), pl.ds(c * HALF, HALF)])
    return carry

  lax.fori_loop(0, NRB, wblock, 0)


@jax.jit
def kernel(h, h_in, edge_index):
  del h_in
  hcat = jnp.concatenate([h[:, :HALF], h[:, HALF:]], axis=0)
  src = edge_index[0].reshape(CHUNKS, CW)
  dst = edge_index[1].reshape(CHUNKS, CW)
  ones = jnp.ones((CW, DW), jnp.float32)
  z64 = jnp.zeros((RPT, HALF), jnp.float32)
  zdeg = jnp.zeros((RPT, DW), jnp.float32)

  mesh = plsc.VectorSubcoreMesh(core_axis_name="c", subcore_axis_name="s")
  run = pl.kernel(
      _body,
      out_type=jax.ShapeDtypeStruct((N, D), jnp.float32),
      mesh=mesh,
      compiler_params=pltpu.CompilerParams(use_tc_tiling_on_sc=False),
      scratch_types=[
          pltpu.VMEM_SHARED((N, HALF), jnp.float32),
          pltpu.VMEM_SHARED((N, DW), jnp.float32),
          pltpu.VMEM((BLK, CW), jnp.int32),
          pltpu.VMEM((BLK, CW), jnp.int32),
          pltpu.VMEM((CW, HALF), jnp.float32),
          pltpu.VMEM((CW, DW), jnp.float32),
          pltpu.VMEM((RB, HALF), jnp.float32),
          pltpu.VMEM((RB, HALF), jnp.float32),
          pltpu.VMEM((RB, HALF), jnp.float32),
          pltpu.VMEM((RB, DW), jnp.float32),
      ],
  )
  return run(hcat, src, dst, ones, z64, zdeg)


# speedup vs baseline: 8.1681x; 1.2649x over previous
import jax
import jax.numpy as jnp
from jax import lax
from jax.experimental import pallas as pl
from jax.experimental.pallas import tpu as pltpu
from jax.experimental.pallas import tpu_sc as plsc

N = 10000
E = 320000
D = 128

NC = 2
NS = 16
HALF = D // NC
CW = 100
CHUNKS = E // CW
CPT = CHUNKS // NS
BLK = 20
NBLK = CPT // BLK
RPT = N // NS
RB = 125
NRB = RPT // RB
DW = 16


def _body(hcat_hbm, src_hbm, dst_hbm, ones_hbm, z64_hbm, zdeg_hbm,
          out_hbm,
          acc, deg,
          src_buf, dst_buf, msgs0, msgs1, ones_t,
          acc_t, h_t, out_t, deg_t, gsem0, gsem1):
  c = lax.axis_index("c")
  s = lax.axis_index("s")
  hview = hcat_hbm.at[pl.ds(c * N, N)]

  r0 = s * RPT
  pltpu.sync_copy(z64_hbm, acc.at[pl.ds(r0, RPT)])
  pltpu.sync_copy(zdeg_hbm, deg.at[pl.ds(r0, RPT)])
  pltpu.sync_copy(ones_hbm, ones_t)
  plsc.subcore_barrier()

  msgs = (msgs0, msgs1)
  gsem = (gsem0, gsem1)

  def block(b, carry):
    base = s * CPT + b * BLK
    pltpu.sync_copy(src_hbm.at[pl.ds(base, BLK)], src_buf)
    pltpu.sync_copy(dst_hbm.at[pl.ds(base, BLK)], dst_buf)
    pltpu.async_copy(hview.at[src_buf.at[0]], msgs[0], gsem[0])
    for j in range(BLK):
      p = j % 2
      pltpu.make_async_copy(hview.at[src_buf.at[j]], msgs[p], gsem[p]).wait()
      if j + 1 < BLK:
        pltpu.async_copy(hview.at[src_buf.at[j + 1]], msgs[1 - p], gsem[1 - p])
      pltpu.sync_copy(msgs[p], acc.at[dst_buf.at[j]], add=True)
      pltpu.sync_copy(ones_t, deg.at[dst_buf.at[j]], add=True)
    return carry

  lax.fori_loop(0, NBLK, block, 0)
  plsc.subcore_barrier()

  def wblock(b, carry):
    row = s * RPT + b * RB
    pltpu.sync_copy(acc.at[pl.ds(row, RB)], acc_t)
    pltpu.sync_copy(hview.at[pl.ds(row, RB)], h_t)
    pltpu.sync_copy(deg.at[pl.ds(row, RB)], deg_t)

    def wrow(i, carry2):
      m = deg_t[i, pl.ds(0, DW)] > 0.0
      for k in range(HALF // 16):
        av = acc_t[i, pl.ds(k * 16, 16)]
        hv = h_t[i, pl.ds(k * 16, 16)]
        out_t[i, pl.ds(k * 16, 16)] = jnp.where(m, av, hv)
      return carry2

    lax.fori_loop(0, RB, wrow, 0)
    pltpu.sync_copy(out_t,
                    out_hbm.at[pl.ds(row, RB), pl.ds(c * HALF, HALF)])
    return carry

  lax.fori_loop(0, NRB, wblock, 0)


@jax.jit
def kernel(h, h_in, edge_index):
  del h_in
  hcat = jnp.concatenate([h[:, :HALF], h[:, HALF:]], axis=0)
  src = edge_index[0].reshape(CHUNKS, CW)
  dst = edge_index[1].reshape(CHUNKS, CW)
  ones = jnp.ones((CW, DW), jnp.float32)
  z64 = jnp.zeros((RPT, HALF), jnp.float32)
  zdeg = jnp.zeros((RPT, DW), jnp.float32)

  mesh = plsc.VectorSubcoreMesh(core_axis_name="c", subcore_axis_name="s")
  run = pl.kernel(
      _body,
      out_type=jax.ShapeDtypeStruct((N, D), jnp.float32),
      mesh=mesh,
      compiler_params=pltpu.CompilerParams(use_tc_tiling_on_sc=False),
      scratch_types=[
          pltpu.VMEM_SHARED((N, HALF), jnp.float32),
          pltpu.VMEM_SHARED((N, DW), jnp.float32),
          pltpu.VMEM((BLK, CW), jnp.int32),
          pltpu.VMEM((BLK, CW), jnp.int32),
          pltpu.VMEM((CW, HALF), jnp.float32),
          pltpu.VMEM((CW, HALF), jnp.float32),
          pltpu.VMEM((CW, DW), jnp.float32),
          pltpu.VMEM((RB, HALF), jnp.float32),
          pltpu.VMEM((RB, HALF), jnp.float32),
          pltpu.VMEM((RB, HALF), jnp.float32),
          pltpu.VMEM((RB, DW), jnp.float32),
          pltpu.SemaphoreType.DMA,
          pltpu.SemaphoreType.DMA,
      ],
  )
  return run(hcat, src, dst, ones, z64, zdeg)
